# baseline (speedup 1.0000x reference)
import functools
import math

import jax
import jax.numpy as jnp
import numpy as np
from jax import lax
from jax.experimental import pallas as pl
from jax.experimental.pallas import tpu as pltpu
from jax.experimental.pallas import tpu_sc as plsc

VOCAB = 100000
D_MODEL = 768
MAX_SEQ_LEN = 2048
BATCH = 4
SEQ_LEN = 2048

NUM_WORKERS = 32
POS_PER_W = SEQ_LEN // NUM_WORKERS
TOK_PER_W = BATCH * POS_PER_W
CHUNK = 16
NCHUNK = TOK_PER_W // CHUNK
HALVES = POS_PER_W // CHUNK
LANES = 16
GROUPS = D_MODEL // LANES
SCALE = math.sqrt(D_MODEL)


def _make_pe_np(max_seq_len, d_model):
    pe = np.zeros((max_seq_len, d_model), dtype=np.float32)
    position = np.arange(0, max_seq_len, dtype=np.float32)[:, None]
    div_term = np.exp(
        np.arange(0, d_model, 2, dtype=np.float32) * (-math.log(10000.0) / d_model)
    )
    pe[:, 0::2] = np.sin(position * div_term)
    pe[:, 1::2] = np.cos(position * div_term)
    return pe


def _pe_packed_np():
    import ml_dtypes
    pe = _make_pe_np(MAX_SEQ_LEN, D_MODEL)
    pe = pe.reshape(MAX_SEQ_LEN, D_MODEL // 32, 2, 16)
    pe = pe.transpose(0, 1, 3, 2).reshape(MAX_SEQ_LEN, D_MODEL)
    pe = np.ascontiguousarray(pe.astype(ml_dtypes.bfloat16))
    return pe.view(np.int32)


_PE = _pe_packed_np()
_PE_DEV = None


def _pe_on_device():
    global _PE_DEV
    if _PE_DEV is None:
        _PE_DEV = jnp.asarray(_PE)
    return _PE_DEV


def _body(ids_hbm, pe_hbm, table_hbm, out_hbm, idx_v, rows_v, pe_v,
          gsem, osem, psem):
    wid = lax.axis_index("s") * 2 + lax.axis_index("c")
    s0 = wid * POS_PER_W

    pe_dma = pltpu.async_copy(pe_hbm.at[pl.ds(s0, POS_PER_W)], pe_v, psem)
    pltpu.sync_copy(ids_hbm.at[0, pl.ds(s0, POS_PER_W)], idx_v.at[0])

    def gather(c, slot):
        b = lax.div(c, HALVES)
        h = lax.rem(c, HALVES)
        idx = idx_v.at[b, pl.ds(h * CHUNK, CHUNK)]
        return pltpu.async_copy(table_hbm.at[idx], rows_v.at[slot],
                                gsem.at[slot])

    def put(c, slot):
        b = lax.div(c, HALVES)
        off = b * SEQ_LEN + s0 + lax.rem(c, HALVES) * CHUNK
        return pltpu.async_copy(rows_v.at[slot],
                                out_hbm.at[pl.ds(off, CHUNK)],
                                osem.at[slot])

    def wait_out(slot):
        pltpu.make_async_copy(rows_v.at[slot],
                              out_hbm.at[pl.ds(0, CHUNK)],
                              osem.at[slot]).wait()

    def wait_gather(slot):
        pltpu.make_async_copy(table_hbm.at[idx_v.at[0, pl.ds(0, CHUNK)]],
                              rows_v.at[slot], gsem.at[slot]).wait()

    def compute(c, slot):
        pbase = lax.rem(c, HALVES) * CHUNK

        @plsc.parallel_loop(0, CHUNK, step=1)
        def _row_body(t):
            for k in range(GROUPS // 2):
                w = pe_v[pbase + t, pl.ds(k * LANES, LANES)]
                pa = lax.bitcast_convert_type(lax.shift_left(w, 16),
                                              jnp.float32)
                pb = lax.bitcast_convert_type(
                    jnp.bitwise_and(w, jnp.int32(-65536)), jnp.float32)
                sa = pl.ds(k * 32, LANES)
                sb = pl.ds(k * 32 + LANES, LANES)
                rows_v[slot, t, sa] = rows_v[slot, t, sa] * SCALE + pa
                rows_v[slot, t, sb] = rows_v[slot, t, sb] * SCALE + pb

    for c in range(4):
        gather(jnp.int32(c), jnp.int32(c))
    for b in range(1, BATCH):
        pltpu.sync_copy(ids_hbm.at[b, pl.ds(s0, POS_PER_W)], idx_v.at[b])
    for c in range(4, 7):
        gather(jnp.int32(c), jnp.int32(c))
    pe_dma.wait()

    def chunk_body(c, carry):
        slot = lax.rem(c, 8)
        nslot = lax.rem(c + 7, 8)

        @pl.when(c + 7 < NCHUNK)
        def _():
            @pl.when(c >= 1)
            def _():
                wait_out(nslot)
            gather(c + 7, nslot)

        wait_gather(slot)
        compute(c, slot)
        put(c, slot)
        return carry

    lax.fori_loop(0, NCHUNK, chunk_body, 0)
    for s in range(8):
        wait_out(s)


@jax.jit
def _combined_embedding(ids3, pe, W):
    mesh = plsc.VectorSubcoreMesh(core_axis_name="c", subcore_axis_name="s",
                                  num_cores=2, num_subcores=16)
    return pl.kernel(
        _body,
        out_type=jax.ShapeDtypeStruct((BATCH * SEQ_LEN, D_MODEL), jnp.float32),
        mesh=mesh,
        scratch_types=[
            pltpu.VMEM((BATCH, POS_PER_W), jnp.int32),
            pltpu.VMEM((8, CHUNK, D_MODEL), jnp.float32),
            pltpu.VMEM((POS_PER_W, D_MODEL // 2), jnp.int32),
            pltpu.SemaphoreType.DMA((8,)),
            pltpu.SemaphoreType.DMA((8,)),
            pltpu.SemaphoreType.DMA,
        ],
    )(ids3, pe, W)


def kernel(token_ids, W):
    out = _combined_embedding(token_ids.astype(jnp.int32), _pe_on_device(), W)
    return out.reshape(BATCH, SEQ_LEN, D_MODEL)


# speedup vs baseline: 1.0096x; 1.0096x over previous
import functools
import math

import jax
import jax.numpy as jnp
import numpy as np
from jax import lax
from jax.experimental import pallas as pl
from jax.experimental.pallas import tpu as pltpu
from jax.experimental.pallas import tpu_sc as plsc

VOCAB = 100000
D_MODEL = 768
MAX_SEQ_LEN = 2048
BATCH = 4
SEQ_LEN = 2048

NUM_WORKERS = 32
POS_PER_W = SEQ_LEN // NUM_WORKERS
TOK_PER_W = BATCH * POS_PER_W
CHUNK = 32
NCHUNK = TOK_PER_W // CHUNK
HALVES = POS_PER_W // CHUNK
LANES = 16
GROUPS = D_MODEL // LANES
SCALE = math.sqrt(D_MODEL)


def _make_pe_np(max_seq_len, d_model):
    pe = np.zeros((max_seq_len, d_model), dtype=np.float32)
    position = np.arange(0, max_seq_len, dtype=np.float32)[:, None]
    div_term = np.exp(
        np.arange(0, d_model, 2, dtype=np.float32) * (-math.log(10000.0) / d_model)
    )
    pe[:, 0::2] = np.sin(position * div_term)
    pe[:, 1::2] = np.cos(position * div_term)
    return pe


def _pe_packed_np():
    import ml_dtypes
    pe = _make_pe_np(MAX_SEQ_LEN, D_MODEL)
    pe = pe.reshape(MAX_SEQ_LEN, D_MODEL // 32, 2, 16)
    pe = pe.transpose(0, 1, 3, 2).reshape(MAX_SEQ_LEN, D_MODEL)
    pe = np.ascontiguousarray(pe.astype(ml_dtypes.bfloat16))
    return pe.view(np.int32)


_PE = _pe_packed_np()
_PE_DEV = None


def _pe_on_device():
    global _PE_DEV
    if _PE_DEV is None:
        _PE_DEV = jnp.asarray(_PE)
    return _PE_DEV


def _body(ids_hbm, pe_hbm, table_hbm, out_hbm, idx_v, rows_v, pe_v,
          gsem, osem, psem, isem):
    wid = lax.axis_index("s") * 2 + lax.axis_index("c")
    s0 = wid * POS_PER_W

    pe_dma = pltpu.async_copy(pe_hbm.at[pl.ds(s0, POS_PER_W)], pe_v, psem)
    pltpu.sync_copy(ids_hbm.at[0, pl.ds(s0, POS_PER_W)], idx_v.at[0])

    def gather(c, slot):
        b = lax.div(c, HALVES)
        h = lax.rem(c, HALVES)
        idx = idx_v.at[b, pl.ds(h * CHUNK, CHUNK)]
        return pltpu.async_copy(table_hbm.at[idx], rows_v.at[slot],
                                gsem.at[slot])

    def put(c, slot):
        b = lax.div(c, HALVES)
        off = b * SEQ_LEN + s0 + lax.rem(c, HALVES) * CHUNK
        return pltpu.async_copy(rows_v.at[slot],
                                out_hbm.at[pl.ds(off, CHUNK)],
                                osem.at[slot])

    def wait_out(slot):
        pltpu.make_async_copy(rows_v.at[slot],
                              out_hbm.at[pl.ds(0, CHUNK)],
                              osem.at[slot]).wait()

    def wait_gather(slot):
        pltpu.make_async_copy(table_hbm.at[idx_v.at[0, pl.ds(0, CHUNK)]],
                              rows_v.at[slot], gsem.at[slot]).wait()

    def compute(c, slot):
        pbase = lax.rem(c, HALVES) * CHUNK

        @plsc.parallel_loop(0, CHUNK, step=1)
        def _row_body(t):
            for k in range(GROUPS // 2):
                w = pe_v[pbase + t, pl.ds(k * LANES, LANES)]
                pa = lax.bitcast_convert_type(lax.shift_left(w, 16),
                                              jnp.float32)
                pb = lax.bitcast_convert_type(
                    jnp.bitwise_and(w, jnp.int32(-65536)), jnp.float32)
                sa = pl.ds(k * 32, LANES)
                sb = pl.ds(k * 32 + LANES, LANES)
                rows_v[slot, t, sa] = rows_v[slot, t, sa] * SCALE + pa
                rows_v[slot, t, sb] = rows_v[slot, t, sb] * SCALE + pb

    gather(jnp.int32(0), jnp.int32(0))
    gather(jnp.int32(1), jnp.int32(1))
    ids_dmas = [
        pltpu.async_copy(ids_hbm.at[b, pl.ds(s0, POS_PER_W)], idx_v.at[b],
                         isem)
        for b in range(1, BATCH)
    ]
    for d in ids_dmas:
        d.wait()
    gather(jnp.int32(2), jnp.int32(2))
    pe_dma.wait()

    def chunk_body(c, carry):
        slot = lax.rem(c, 4)
        nslot = lax.rem(c + 3, 4)

        @pl.when(c + 3 < NCHUNK)
        def _():
            @pl.when(c >= 1)
            def _():
                wait_out(nslot)
            gather(c + 3, nslot)

        wait_gather(slot)
        compute(c, slot)
        put(c, slot)
        return carry

    lax.fori_loop(0, NCHUNK, chunk_body, 0)
    wait_out(0)
    wait_out(1)
    wait_out(2)
    wait_out(3)


@jax.jit
def _combined_embedding(ids3, pe, W):
    mesh = plsc.VectorSubcoreMesh(core_axis_name="c", subcore_axis_name="s",
                                  num_cores=2, num_subcores=16)
    return pl.kernel(
        _body,
        out_type=jax.ShapeDtypeStruct((BATCH * SEQ_LEN, D_MODEL), jnp.float32),
        mesh=mesh,
        scratch_types=[
            pltpu.VMEM((BATCH, POS_PER_W), jnp.int32),
            pltpu.VMEM((4, CHUNK, D_MODEL), jnp.float32),
            pltpu.VMEM((POS_PER_W, D_MODEL // 2), jnp.int32),
            pltpu.SemaphoreType.DMA((4,)),
            pltpu.SemaphoreType.DMA((4,)),
            pltpu.SemaphoreType.DMA,
            pltpu.SemaphoreType.DMA,
        ],
    )(ids3, pe, W)


def kernel(token_ids, W):
    out = _combined_embedding(token_ids.astype(jnp.int32), _pe_on_device(), W)
    return out.reshape(BATCH, SEQ_LEN, D_MODEL)


# speedup vs baseline: 1.0362x; 1.0264x over previous
import functools
import math

import jax
import jax.numpy as jnp
import numpy as np
from jax import lax
from jax.experimental import pallas as pl
from jax.experimental.pallas import tpu as pltpu
from jax.experimental.pallas import tpu_sc as plsc

VOCAB = 100000
D_MODEL = 768
MAX_SEQ_LEN = 2048
BATCH = 4
SEQ_LEN = 2048

NUM_WORKERS = 32
POS_PER_W = SEQ_LEN // NUM_WORKERS
TOK_PER_W = BATCH * POS_PER_W
CHUNK = 32
NCHUNK = TOK_PER_W // CHUNK
HALVES = POS_PER_W // CHUNK
LANES = 16
GROUPS = D_MODEL // LANES
SCALE = math.sqrt(D_MODEL)


def _make_pe_np(max_seq_len, d_model):
    pe = np.zeros((max_seq_len, d_model), dtype=np.float32)
    position = np.arange(0, max_seq_len, dtype=np.float32)[:, None]
    div_term = np.exp(
        np.arange(0, d_model, 2, dtype=np.float32) * (-math.log(10000.0) / d_model)
    )
    pe[:, 0::2] = np.sin(position * div_term)
    pe[:, 1::2] = np.cos(position * div_term)
    return pe


def _pe_packed_np():
    import ml_dtypes
    pe = _make_pe_np(MAX_SEQ_LEN, D_MODEL)
    pe = pe.reshape(MAX_SEQ_LEN, D_MODEL // 32, 2, 16)
    pe = pe.transpose(0, 1, 3, 2).reshape(MAX_SEQ_LEN, D_MODEL)
    pe = np.ascontiguousarray(pe.astype(ml_dtypes.bfloat16))
    return pe.view(np.int32)


_PE = _pe_packed_np()
_PE_DEV = None


def _pe_on_device():
    global _PE_DEV
    if _PE_DEV is None:
        _PE_DEV = jnp.asarray(_PE)
    return _PE_DEV


def _body(ids_hbm, pe_hbm, table_hbm, out_hbm, idx_v, rows_v, pe_v,
          gsem, osem, psem):
    wid = lax.axis_index("s") * 2 + lax.axis_index("c")
    s0 = wid * POS_PER_W

    pe_dma = pltpu.async_copy(pe_hbm.at[pl.ds(s0, POS_PER_W)], pe_v, psem)
    pltpu.sync_copy(ids_hbm.at[0, pl.ds(s0, POS_PER_W)], idx_v.at[0])

    def gather(c, slot):
        b = lax.div(c, HALVES)
        h = lax.rem(c, HALVES)
        idx = idx_v.at[b, pl.ds(h * CHUNK, CHUNK)]
        return pltpu.async_copy(table_hbm.at[idx], rows_v.at[slot],
                                gsem.at[slot])

    def put(c, slot):
        b = lax.div(c, HALVES)
        off = b * SEQ_LEN + s0 + lax.rem(c, HALVES) * CHUNK
        return pltpu.async_copy(rows_v.at[slot],
                                out_hbm.at[pl.ds(off, CHUNK)],
                                osem.at[slot])

    def wait_out(slot):
        pltpu.make_async_copy(rows_v.at[slot],
                              out_hbm.at[pl.ds(0, CHUNK)],
                              osem.at[slot]).wait()

    def wait_gather(slot):
        pltpu.make_async_copy(table_hbm.at[idx_v.at[0, pl.ds(0, CHUNK)]],
                              rows_v.at[slot], gsem.at[slot]).wait()

    def compute(c, slot):
        pbase = lax.rem(c, HALVES) * CHUNK

        @plsc.parallel_loop(0, CHUNK, step=1)
        def _row_body(t):
            for k in range(GROUPS // 2):
                w = pe_v[pbase + t, pl.ds(k * LANES, LANES)]
                pa = lax.bitcast_convert_type(lax.shift_left(w, 16),
                                              jnp.float32)
                pb = lax.bitcast_convert_type(
                    jnp.bitwise_and(w, jnp.int32(-65536)), jnp.float32)
                sa = pl.ds(k * 32, LANES)
                sb = pl.ds(k * 32 + LANES, LANES)
                rows_v[slot, t, sa] = rows_v[slot, t, sa] * SCALE + pa
                rows_v[slot, t, sb] = rows_v[slot, t, sb] * SCALE + pb

    gather(jnp.int32(0), jnp.int32(0))
    gather(jnp.int32(1), jnp.int32(1))
    for b in range(1, BATCH):
        pltpu.sync_copy(ids_hbm.at[b, pl.ds(s0, POS_PER_W)], idx_v.at[b])
    gather(jnp.int32(2), jnp.int32(2))
    pe_dma.wait()

    def chunk_body(c, carry):
        slot = lax.rem(c, 4)
        nslot = lax.rem(c + 3, 4)

        @pl.when(c + 3 < NCHUNK)
        def _():
            @pl.when(c >= 1)
            def _():
                wait_out(nslot)
            gather(c + 3, nslot)

        wait_gather(slot)
        compute(c, slot)
        put(c, slot)
        return carry

    lax.fori_loop(0, NCHUNK, chunk_body, 0)
    wait_out(0)
    wait_out(1)
    wait_out(2)
    wait_out(3)


@jax.jit
def _combined_embedding(ids3, pe, W):
    mesh = plsc.VectorSubcoreMesh(core_axis_name="c", subcore_axis_name="s",
                                  num_cores=2, num_subcores=16)
    return pl.kernel(
        _body,
        out_type=jax.ShapeDtypeStruct((BATCH * SEQ_LEN, D_MODEL), jnp.float32),
        mesh=mesh,
        scratch_types=[
            pltpu.VMEM((BATCH, POS_PER_W), jnp.int32),
            pltpu.VMEM((4, CHUNK, D_MODEL), jnp.float32),
            pltpu.VMEM((POS_PER_W, D_MODEL // 2), jnp.int32),
            pltpu.SemaphoreType.DMA((4,)),
            pltpu.SemaphoreType.DMA((4,)),
            pltpu.SemaphoreType.DMA,
        ],
    )(ids3, pe, W)


def kernel(token_ids, W):
    out = _combined_embedding(token_ids.astype(jnp.int32), _pe_on_device(), W)
    return out.reshape(BATCH, SEQ_LEN, D_MODEL)
